# baseline (speedup 1.0000x reference)
import functools

import jax
import jax.numpy as jnp
from jax import lax
from jax.experimental import pallas as pl
from jax.experimental.pallas import tpu as pltpu
from jax.experimental.pallas import tpu_sc as plsc

N_PAD = 10240
NC, NS, L = 2, 16, 16
NW = NC * NS
K = 128


def _mesh():
    return plsc.VectorSubcoreMesh(
        core_axis_name="c", subcore_axis_name="s", num_cores=NC, num_subcores=NS)


def _make_deg(nc0, nc1):
    assert nc0 % 2 == 0 and nc1 % 2 == 0
    ncmax = max(nc0, nc1)

    @functools.partial(
        pl.kernel,
        out_type=jax.ShapeDtypeStruct((NC, N_PAD, L), jnp.float32),
        mesh=_mesh(),
        compiler_params=pltpu.CompilerParams(use_tc_tiling_on_sc=False),
        scratch_types=[
            pltpu.VMEM_SHARED((N_PAD, L), jnp.float32),
            pltpu.VMEM((ncmax, K), jnp.int32),
            pltpu.VMEM((ncmax, K), jnp.float32),
            pltpu.VMEM((K, L), jnp.float32),
            pltpu.VMEM((K, L), jnp.float32),
            pltpu.SemaphoreType.DMA,
            pltpu.SemaphoreType.DMA,
        ],
    )
    def k(col_hbm, ew_hbm, zeros_hbm, out_hbm,
          acc, col_b, w_b, wbuf0, wbuf1, ssem0, ssem1):
        cid = lax.axis_index("c")
        sid = lax.axis_index("s")
        wid = cid * NS + sid
        nc_self = jnp.where(cid == 0, nc0, nc1)
        rpt = N_PAD // NS
        pltpu.sync_copy(zeros_hbm.at[pl.ds(sid * rpt, rpt)],
                        acc.at[pl.ds(sid * rpt, rpt)])
        pltpu.sync_copy(col_hbm.at[wid], col_b)
        pltpu.sync_copy(ew_hbm.at[wid], w_b)
        plsc.subcore_barrier()

        def build(wbuf, c):
            for g in range(K // L):
                w16 = w_b[c, pl.ds(g * L, L)]
                for j in range(L):
                    wbuf[g * L + j, pl.ds(0, L)] = jnp.full((L,), w16[j],
                                                            jnp.float32)

        def scatter(c, wbuf, sem):
            pltpu.async_copy(wbuf, acc.at[col_b.at[c]], sem, add=True)

        def wait_scatter(c, wbuf, sem):
            pltpu.make_async_copy(wbuf, acc.at[col_b.at[c]], sem).wait()

        def body(s, carry):
            c0 = 2 * s
            c1 = 2 * s + 1

            @pl.when(s > 0)
            def _():
                wait_scatter(c0 - 2, wbuf0, ssem0)

            build(wbuf0, c0)
            scatter(c0, wbuf0, ssem0)

            @pl.when(s > 0)
            def _():
                wait_scatter(c1 - 2, wbuf1, ssem1)

            build(wbuf1, c1)
            scatter(c1, wbuf1, ssem1)
            return carry

        lax.fori_loop(0, nc_self // 2, body, 0)
        wait_scatter(nc_self - 2, wbuf0, ssem0)
        wait_scatter(nc_self - 1, wbuf1, ssem1)
        plsc.subcore_barrier()
        pltpu.sync_copy(acc.at[pl.ds(sid * rpt, rpt)],
                        out_hbm.at[cid, pl.ds(sid * rpt, rpt)])

    return k


def _make_agg(F, nc0, nc1):
    NB = 4
    assert nc0 % NB == 0 and nc1 % NB == 0
    ncmax = max(nc0, nc1)

    @functools.partial(
        pl.kernel,
        out_type=jax.ShapeDtypeStruct((NC, N_PAD, F), jnp.float32),
        mesh=_mesh(),
        compiler_params=pltpu.CompilerParams(use_tc_tiling_on_sc=False),
        scratch_types=[
            pltpu.VMEM_SHARED((N_PAD, F), jnp.float32),
            pltpu.VMEM((ncmax, K), jnp.int32),
            pltpu.VMEM((ncmax, K), jnp.int32),
            pltpu.VMEM((ncmax, K), jnp.float32),
            [pltpu.VMEM((K, F), jnp.float32)] * NB,
            [pltpu.SemaphoreType.DMA] * NB,
            [pltpu.SemaphoreType.DMA] * NB,
        ],
    )
    def k(row_hbm, col_hbm, ew_hbm, g_hbm, zeros_hbm, out_hbm,
          acc, row_b, col_b, w_b, rbufs, gsems, ssems):
        cid = lax.axis_index("c")
        sid = lax.axis_index("s")
        wid = cid * NS + sid
        nc_self = jnp.where(cid == 0, nc0, nc1)
        rpt = N_PAD // NS
        pltpu.sync_copy(zeros_hbm.at[pl.ds(sid * rpt, rpt)],
                        acc.at[pl.ds(sid * rpt, rpt)])
        pltpu.sync_copy(row_hbm.at[wid], row_b)
        pltpu.sync_copy(col_hbm.at[wid], col_b)
        pltpu.sync_copy(ew_hbm.at[wid], w_b)
        plsc.subcore_barrier()

        def mult(rbuf, c):
            for g in range(K // L):
                w16 = w_b[c, pl.ds(g * L, L)]
                for j in range(L):
                    e = g * L + j
                    for f0 in range(0, F, L):
                        rbuf[e, pl.ds(f0, L)] = rbuf[e, pl.ds(f0, L)] * w16[j]

        def gather(c, rbuf, sem):
            pltpu.async_copy(g_hbm.at[row_b.at[c]], rbuf, sem)

        def wait_gather(c, rbuf, sem):
            pltpu.make_async_copy(g_hbm.at[row_b.at[c]], rbuf, sem).wait()

        def scatter(c, rbuf, sem):
            pltpu.async_copy(rbuf, acc.at[col_b.at[c]], sem, add=True)

        def wait_scatter(c, rbuf, sem):
            pltpu.make_async_copy(rbuf, acc.at[col_b.at[c]], sem).wait()

        for t in range(NB - 1):
            gather(t, rbufs[t], gsems[t])

        def body(s, carry):
            for t in range(NB):
                c = NB * s + t
                wait_gather(c, rbufs[t], gsems[t])
                tp = (t + NB - 1) % NB

                @pl.when(c > 0)
                def _():
                    wait_scatter(c - 1, rbufs[tp], ssems[tp])

                @pl.when(c + NB - 1 < nc_self)
                def _():
                    gather(c + NB - 1, rbufs[tp], gsems[tp])

                mult(rbufs[t], c)
                scatter(c, rbufs[t], ssems[t])
            return carry

        lax.fori_loop(0, nc_self // NB, body, 0)
        wait_scatter(nc_self - 1, rbufs[NB - 1], ssems[NB - 1])
        plsc.subcore_barrier()
        pltpu.sync_copy(acc.at[pl.ds(sid * rpt, rpt)],
                        out_hbm.at[cid, pl.ds(sid * rpt, rpt)])

    return k


def _tc1(x, W1, degp):
    n, _ = x.shape
    h = W1.shape[1]

    def body(x_ref, w_ref, degp_ref, h_ref, g_ref, dinv_ref):
        deg = degp_ref[0, :, 0:1] + degp_ref[1, :, 0:1] + 1.0
        dinv_full = jnp.where(deg > 0, lax.rsqrt(jnp.maximum(deg, 1e-12)), 0.0)
        dinv = dinv_full[:n]
        hm = jnp.dot(x_ref[...], w_ref[...], preferred_element_type=jnp.float32)
        h_ref[...] = hm
        g_ref[...] = hm * dinv
        dinv_ref[...] = dinv

    return pl.pallas_call(
        body,
        out_shape=[
            jax.ShapeDtypeStruct((n, h), jnp.float32),
            jax.ShapeDtypeStruct((n, h), jnp.float32),
            jax.ShapeDtypeStruct((n, 1), jnp.float32),
        ],
    )(x, W1, degp)


def _tc2(accp, h, dinv, b, W2):
    n, _ = h.shape
    h2 = W2.shape[1]

    def body(accp_ref, h_ref, dinv_ref, b_ref, w_ref, hd2_ref, g2_ref):
        a = accp_ref[...]
        agg = a[0, :n] + a[1, :n]
        di = dinv_ref[...]
        pre = di * agg + (di * di) * h_ref[...] + b_ref[...]
        h1 = jnp.maximum(pre, 0.0)
        hd2 = jnp.dot(h1, w_ref[...], preferred_element_type=jnp.float32)
        hd2_ref[...] = hd2
        g2_ref[...] = hd2 * di

    return pl.pallas_call(
        body,
        out_shape=[
            jax.ShapeDtypeStruct((n, h2), jnp.float32),
            jax.ShapeDtypeStruct((n, h2), jnp.float32),
        ],
    )(accp, h, dinv, b, W2)


def _tc3(accp, hd2, dinv, b, Wout, bout):
    n, _ = hd2.shape

    def body(accp_ref, hd2_ref, dinv_ref, b_ref, w_ref, bout_ref, out_ref):
        a = accp_ref[...]
        agg = a[0, :n] + a[1, :n]
        di = dinv_ref[...]
        pre = di * agg + (di * di) * hd2_ref[...] + b_ref[...]
        h2 = jnp.maximum(pre, 0.0)
        z = jnp.dot(h2, w_ref[...], preferred_element_type=jnp.float32) + bout_ref[...]
        out_ref[...] = 1.0 / (1.0 + jnp.exp(-z))

    return pl.pallas_call(
        body,
        out_shape=jax.ShapeDtypeStruct((n, 1), jnp.float32),
    )(accp, hd2, dinv, b, Wout, bout)


def _counts(e_total, frac0, mod):
    tot = (e_total + NS * K - 1) // (NS * K)
    nc0 = max(mod, int(round(tot * frac0 / mod)) * mod)
    nc1 = max(mod, ((tot - nc0 + mod - 1) // mod) * mod)
    return nc0, nc1


def _layout(arrs, nc0, nc1):
    e0 = NS * nc0 * K
    e1 = NS * nc1 * K
    ncm = max(nc0, nc1)
    outs = []
    for a in arrs:
        a0 = a[:e0].reshape(NS, nc0, K)
        tail = a[e0:]
        pad = e0 + e1 - a.shape[0]
        if pad:
            tail = jnp.concatenate([tail, jnp.zeros((pad,), a.dtype)])
        a1 = tail.reshape(NS, nc1, K)
        if nc0 < ncm:
            a0 = jnp.concatenate(
                [a0, jnp.zeros((NS, ncm - nc0, K), a.dtype)], axis=1)
        if nc1 < ncm:
            a1 = jnp.concatenate(
                [a1, jnp.zeros((NS, ncm - nc1, K), a.dtype)], axis=1)
        outs.append(jnp.concatenate([a0, a1], axis=0))
    return outs


FRAC_DEG = 0.56
FRAC_A32 = 0.951
FRAC_A16 = 0.868


def kernel(x, edge_index, edge_weight, W1, b1, W2, b2, Wout, bout):
    e_total = edge_weight.shape[0]
    row = edge_index[0].astype(jnp.int32)
    col = edge_index[1].astype(jnp.int32)
    ew = edge_weight.astype(jnp.float32)

    d0, d1 = _counts(e_total, FRAC_DEG, 2)
    a0, a1 = _counts(e_total, FRAC_A32, 4)
    c0, c1 = _counts(e_total, FRAC_A16, 4)
    col_d, ew_d = _layout([col, ew], d0, d1)
    row_a, col_a, ew_a = _layout([row, col, ew], a0, a1)
    row_c, col_c, ew_c = _layout([row, col, ew], c0, c1)

    h1n = W1.shape[1]
    h2n = W2.shape[1]
    zeros1 = jnp.zeros((N_PAD, L), jnp.float32)
    zeros_a = jnp.zeros((N_PAD, h1n), jnp.float32)
    zeros_b = jnp.zeros((N_PAD, h2n), jnp.float32)

    degp = _make_deg(d0, d1)(col_d, ew_d, zeros1)
    h, g1, dinv = _tc1(x, W1, degp)
    acc1 = _make_agg(h1n, a0, a1)(row_a, col_a, ew_a, g1, zeros_a)
    hd2, g2 = _tc2(acc1, h, dinv, b1.reshape(1, h1n), W2)
    acc2 = _make_agg(h2n, c0, c1)(row_c, col_c, ew_c, g2, zeros_b)
    return _tc3(acc2, hd2, dinv, b2.reshape(1, h2n), Wout, bout.reshape(1, 1))


# speedup vs baseline: 1.0505x; 1.0505x over previous
import functools

import jax
import jax.numpy as jnp
from jax import lax
from jax.experimental import pallas as pl
from jax.experimental.pallas import tpu as pltpu
from jax.experimental.pallas import tpu_sc as plsc

N_PAD = 10240
NC, NS, L = 2, 16, 16
NW = NC * NS
K = 128


def _mesh():
    return plsc.VectorSubcoreMesh(
        core_axis_name="c", subcore_axis_name="s", num_cores=NC, num_subcores=NS)


def _make_deg(nc0, nc1):
    assert nc0 % 2 == 0 and nc1 % 2 == 0
    ncmax = max(nc0, nc1)

    @functools.partial(
        pl.kernel,
        out_type=jax.ShapeDtypeStruct((NC, N_PAD, L), jnp.float32),
        mesh=_mesh(),
        compiler_params=pltpu.CompilerParams(use_tc_tiling_on_sc=False),
        scratch_types=[
            pltpu.VMEM_SHARED((N_PAD, L), jnp.float32),
            pltpu.VMEM((ncmax, K), jnp.int32),
            pltpu.VMEM((ncmax, K), jnp.float32),
            pltpu.VMEM((K, L), jnp.float32),
            pltpu.VMEM((K, L), jnp.float32),
            pltpu.SemaphoreType.DMA,
            pltpu.SemaphoreType.DMA,
        ],
    )
    def k(col_hbm, ew_hbm, zeros_hbm, out_hbm,
          acc, col_b, w_b, wbuf0, wbuf1, ssem0, ssem1):
        cid = lax.axis_index("c")
        sid = lax.axis_index("s")
        nc_self = jnp.where(cid == 0, nc0, nc1)
        rpt = N_PAD // NS
        pltpu.sync_copy(zeros_hbm.at[pl.ds(sid * rpt, rpt)],
                        acc.at[pl.ds(sid * rpt, rpt)])

        @pl.when(cid == 0)
        def _():
            s0 = sid * nc0
            pltpu.sync_copy(col_hbm.at[pl.ds(s0, nc0)], col_b.at[pl.ds(0, nc0)])
            pltpu.sync_copy(ew_hbm.at[pl.ds(s0, nc0)], w_b.at[pl.ds(0, nc0)])

        @pl.when(cid != 0)
        def _():
            s1 = NS * nc0 + sid * nc1
            pltpu.sync_copy(col_hbm.at[pl.ds(s1, nc1)], col_b.at[pl.ds(0, nc1)])
            pltpu.sync_copy(ew_hbm.at[pl.ds(s1, nc1)], w_b.at[pl.ds(0, nc1)])

        plsc.subcore_barrier()

        def build(wbuf, c):
            for g in range(K // L):
                w16 = w_b[c, pl.ds(g * L, L)]
                for j in range(L):
                    wbuf[g * L + j, pl.ds(0, L)] = jnp.full((L,), w16[j],
                                                            jnp.float32)

        def scatter(c, wbuf, sem):
            pltpu.async_copy(wbuf, acc.at[col_b.at[c]], sem, add=True)

        def wait_scatter(c, wbuf, sem):
            pltpu.make_async_copy(wbuf, acc.at[col_b.at[c]], sem).wait()

        def body(s, carry):
            c0 = 2 * s
            c1 = 2 * s + 1

            @pl.when(s > 0)
            def _():
                wait_scatter(c0 - 2, wbuf0, ssem0)

            build(wbuf0, c0)
            scatter(c0, wbuf0, ssem0)

            @pl.when(s > 0)
            def _():
                wait_scatter(c1 - 2, wbuf1, ssem1)

            build(wbuf1, c1)
            scatter(c1, wbuf1, ssem1)
            return carry

        lax.fori_loop(0, nc_self // 2, body, 0)
        wait_scatter(nc_self - 2, wbuf0, ssem0)
        wait_scatter(nc_self - 1, wbuf1, ssem1)
        plsc.subcore_barrier()
        pltpu.sync_copy(acc.at[pl.ds(sid * rpt, rpt)],
                        out_hbm.at[cid, pl.ds(sid * rpt, rpt)])

    return k


def _make_agg(F, nc0, nc1):
    NB = 4
    assert nc0 % NB == 0 and nc1 % NB == 0
    ncmax = max(nc0, nc1)

    @functools.partial(
        pl.kernel,
        out_type=jax.ShapeDtypeStruct((NC, N_PAD, F), jnp.float32),
        mesh=_mesh(),
        compiler_params=pltpu.CompilerParams(use_tc_tiling_on_sc=False),
        scratch_types=[
            pltpu.VMEM_SHARED((N_PAD, F), jnp.float32),
            pltpu.VMEM((ncmax, K), jnp.int32),
            pltpu.VMEM((ncmax, K), jnp.int32),
            pltpu.VMEM((ncmax, K), jnp.float32),
            [pltpu.VMEM((K, F), jnp.float32)] * NB,
            [pltpu.SemaphoreType.DMA] * NB,
            [pltpu.SemaphoreType.DMA] * NB,
        ],
    )
    def k(row_hbm, col_hbm, ew_hbm, g_hbm, zeros_hbm, out_hbm,
          acc, row_b, col_b, w_b, rbufs, gsems, ssems):
        cid = lax.axis_index("c")
        sid = lax.axis_index("s")
        nc_self = jnp.where(cid == 0, nc0, nc1)
        rpt = N_PAD // NS
        pltpu.sync_copy(zeros_hbm.at[pl.ds(sid * rpt, rpt)],
                        acc.at[pl.ds(sid * rpt, rpt)])

        @pl.when(cid == 0)
        def _():
            s0 = sid * nc0
            pltpu.sync_copy(row_hbm.at[pl.ds(s0, nc0)], row_b.at[pl.ds(0, nc0)])
            pltpu.sync_copy(col_hbm.at[pl.ds(s0, nc0)], col_b.at[pl.ds(0, nc0)])
            pltpu.sync_copy(ew_hbm.at[pl.ds(s0, nc0)], w_b.at[pl.ds(0, nc0)])

        @pl.when(cid != 0)
        def _():
            s1 = NS * nc0 + sid * nc1
            pltpu.sync_copy(row_hbm.at[pl.ds(s1, nc1)], row_b.at[pl.ds(0, nc1)])
            pltpu.sync_copy(col_hbm.at[pl.ds(s1, nc1)], col_b.at[pl.ds(0, nc1)])
            pltpu.sync_copy(ew_hbm.at[pl.ds(s1, nc1)], w_b.at[pl.ds(0, nc1)])

        plsc.subcore_barrier()

        def mult(rbuf, c):
            for g in range(K // L):
                w16 = w_b[c, pl.ds(g * L, L)]
                for j in range(L):
                    e = g * L + j
                    for f0 in range(0, F, L):
                        rbuf[e, pl.ds(f0, L)] = rbuf[e, pl.ds(f0, L)] * w16[j]

        def gather(c, rbuf, sem):
            pltpu.async_copy(g_hbm.at[row_b.at[c]], rbuf, sem)

        def wait_gather(c, rbuf, sem):
            pltpu.make_async_copy(g_hbm.at[row_b.at[c]], rbuf, sem).wait()

        def scatter(c, rbuf, sem):
            pltpu.async_copy(rbuf, acc.at[col_b.at[c]], sem, add=True)

        def wait_scatter(c, rbuf, sem):
            pltpu.make_async_copy(rbuf, acc.at[col_b.at[c]], sem).wait()

        for t in range(NB - 1):
            gather(t, rbufs[t], gsems[t])

        def body(s, carry):
            for t in range(NB):
                c = NB * s + t
                wait_gather(c, rbufs[t], gsems[t])
                tp = (t + NB - 1) % NB

                @pl.when(c > 0)
                def _():
                    wait_scatter(c - 1, rbufs[tp], ssems[tp])

                @pl.when(c + NB - 1 < nc_self)
                def _():
                    gather(c + NB - 1, rbufs[tp], gsems[tp])

                mult(rbufs[t], c)
                scatter(c, rbufs[t], ssems[t])
            return carry

        lax.fori_loop(0, nc_self // NB, body, 0)
        wait_scatter(nc_self - 1, rbufs[NB - 1], ssems[NB - 1])
        plsc.subcore_barrier()
        pltpu.sync_copy(acc.at[pl.ds(sid * rpt, rpt)],
                        out_hbm.at[cid, pl.ds(sid * rpt, rpt)])

    return k


def _tc_mm(x, W):
    n = x.shape[0]
    h = W.shape[1]

    def body(x_ref, w_ref, h_ref):
        h_ref[...] = jnp.dot(x_ref[...], w_ref[...],
                             preferred_element_type=jnp.float32)

    return pl.pallas_call(
        body, out_shape=jax.ShapeDtypeStruct((n, h), jnp.float32))(x, W)


def _tc1b(h, degp):
    n, hw = h.shape

    def body(h_ref, degp_ref, g_ref, dinv_ref):
        deg = degp_ref[0, :, 0:1] + degp_ref[1, :, 0:1] + 1.0
        dinv_full = jnp.where(deg > 0, lax.rsqrt(jnp.maximum(deg, 1e-12)), 0.0)
        dinv = dinv_full[:n]
        g_ref[...] = h_ref[...] * dinv
        dinv_ref[...] = dinv

    return pl.pallas_call(
        body,
        out_shape=[
            jax.ShapeDtypeStruct((n, hw), jnp.float32),
            jax.ShapeDtypeStruct((n, 1), jnp.float32),
        ],
    )(h, degp)


def _tc2(accp, h, dinv, b, W2):
    n, _ = h.shape
    h2 = W2.shape[1]

    def body(accp_ref, h_ref, dinv_ref, b_ref, w_ref, hd2_ref, g2_ref):
        a = accp_ref[...]
        agg = a[0, :n] + a[1, :n]
        di = dinv_ref[...]
        pre = di * agg + (di * di) * h_ref[...] + b_ref[...]
        h1 = jnp.maximum(pre, 0.0)
        hd2 = jnp.dot(h1, w_ref[...], preferred_element_type=jnp.float32)
        hd2_ref[...] = hd2
        g2_ref[...] = hd2 * di

    return pl.pallas_call(
        body,
        out_shape=[
            jax.ShapeDtypeStruct((n, h2), jnp.float32),
            jax.ShapeDtypeStruct((n, h2), jnp.float32),
        ],
    )(accp, h, dinv, b, W2)


def _tc3(accp, hd2, dinv, b, Wout, bout):
    n, _ = hd2.shape

    def body(accp_ref, hd2_ref, dinv_ref, b_ref, w_ref, bout_ref, out_ref):
        a = accp_ref[...]
        agg = a[0, :n] + a[1, :n]
        di = dinv_ref[...]
        pre = di * agg + (di * di) * hd2_ref[...] + b_ref[...]
        h2 = jnp.maximum(pre, 0.0)
        z = jnp.dot(h2, w_ref[...], preferred_element_type=jnp.float32) + bout_ref[...]
        out_ref[...] = 1.0 / (1.0 + jnp.exp(-z))

    return pl.pallas_call(
        body,
        out_shape=jax.ShapeDtypeStruct((n, 1), jnp.float32),
    )(accp, hd2, dinv, b, Wout, bout)


def _layout(arrs, nc0, nc1):
    tot = NS * (nc0 + nc1) * K
    return [jnp.pad(a, (0, tot - a.shape[0])).reshape(-1, K) for a in arrs]


def _tc1(x, W1, degp):
    h = _tc_mm(x, W1)
    g1, dinv = _tc1b(h, degp)
    return h, g1, dinv


def _counts(e_total, frac0, mod):
    tot = (e_total + NS * K - 1) // (NS * K)
    nc0 = max(mod, int(round(tot * frac0 / mod)) * mod)
    nc1 = max(mod, ((tot - nc0 + mod - 1) // mod) * mod)
    return nc0, nc1


FRAC_DEG = 0.56
FRAC_A32 = 0.951
FRAC_A16 = 0.868


def kernel(x, edge_index, edge_weight, W1, b1, W2, b2, Wout, bout):
    e_total = edge_weight.shape[0]
    row = edge_index[0].astype(jnp.int32)
    col = edge_index[1].astype(jnp.int32)
    ew = edge_weight.astype(jnp.float32)

    d0, d1 = _counts(e_total, FRAC_DEG, 2)
    a0, a1 = _counts(e_total, FRAC_A32, 4)
    c0, c1 = _counts(e_total, FRAC_A16, 4)
    col_d, ew_d = _layout([col, ew], d0, d1)
    row_a, col_a, ew_a = _layout([row, col, ew], a0, a1)
    row_c, col_c, ew_c = _layout([row, col, ew], c0, c1)

    h1n = W1.shape[1]
    h2n = W2.shape[1]
    zeros1 = jnp.zeros((N_PAD, L), jnp.float32)
    zeros_a = jnp.zeros((N_PAD, h1n), jnp.float32)
    zeros_b = jnp.zeros((N_PAD, h2n), jnp.float32)

    degp = _make_deg(d0, d1)(col_d, ew_d, zeros1)
    h, g1, dinv = _tc1(x, W1, degp)
    acc1 = _make_agg(h1n, a0, a1)(row_a, col_a, ew_a, g1, zeros_a)
    hd2, g2 = _tc2(acc1, h, dinv, b1.reshape(1, h1n), W2)
    acc2 = _make_agg(h2n, c0, c1)(row_c, col_c, ew_c, g2, zeros_b)
    return _tc3(acc2, hd2, dinv, b2.reshape(1, h2n), Wout, bout.reshape(1, 1))
